# baseline (device time: 130956 ns/iter reference)
import jax
import jax.numpy as jnp
from jax import lax
from jax.experimental import pallas as pl
from jax.experimental.pallas import tpu as pltpu

N_DEV = 4
N_EXPERTS = 16
EPD = N_EXPERTS // N_DEV
CAP = 160
BLOCK = EPD * CAP


def _a2a_moe_a2a(xb, slot_row, slot_col, W1, W2):
    T, D = xb.shape
    F = W1.shape[2]

    def body(x_ref, srow_ref, scol_ref, w1_hbm, w2_hbm, out_ref,
             send_ref, recv_ref, w1b_ref, w2b_ref, stage_ref,
             s1, r1, s2, r2, csem):
        me = lax.axis_index("i")

        barrier = pltpu.get_barrier_semaphore()
        for j in range(1, N_DEV):
            pl.semaphore_signal(
                barrier, inc=1,
                device_id=((me + j) % N_DEV,),
                device_id_type=pl.DeviceIdType.MESH,
            )

        iota_r = lax.broadcasted_iota(jnp.int32, (BLOCK, 1), 0)

        def pack(dst):
            sd = (srow_ref[...] - dst * BLOCK == iota_r).astype(jnp.bfloat16)
            return jnp.dot(
                sd, x_ref[...], preferred_element_type=jnp.float32
            ).astype(jnp.bfloat16)

        pl.semaphore_wait(barrier, N_DEV - 1)
        p1 = []
        for j in (2, 1, 3):
            p = (me + j) % N_DEV
            send_ref[p] = pack(p)
            rdma = pltpu.make_async_remote_copy(
                src_ref=send_ref.at[p],
                dst_ref=recv_ref.at[me],
                send_sem=s1.at[p],
                recv_sem=r1.at[me],
                device_id=(p,),
                device_id_type=pl.DeviceIdType.MESH,
            )
            rdma.start()
            p1.append(rdma)

        recv_ref[me] = pack(me)

        n_chunks = EPD * 4

        def make_cast_loop(w_hbm, wb_ref, w1_layout):
            def slices(i):
                k = i // 4
                if w1_layout:
                    rs = pl.ds(((i // 2) % 2) * (D // 2), D // 2)
                    cs = pl.ds((i % 2) * (F // 2), F // 2)
                else:
                    rs = pl.ds((i % 4) * (F // 4), F // 4)
                    cs = pl.ds(0, D)
                return k, rs, cs

            def chunk_copy(i):
                k, rs, cs = slices(i)
                return pltpu.make_async_copy(
                    w_hbm.at[k, rs, cs], stage_ref.at[i % 2], csem.at[i % 2]
                )

            def step(i, carry):
                @pl.when(i + 1 < n_chunks)
                def _():
                    chunk_copy(i + 1).start()
                chunk_copy(i).wait()
                k, rs, cs = slices(i)
                wb_ref[k, rs, cs] = stage_ref[i % 2].astype(jnp.bfloat16)
                return carry

            chunk_copy(0).start()
            lax.fori_loop(0, n_chunks, step, 0)

        make_cast_loop(w1_hbm, w1b_ref, True)
        make_cast_loop(w2_hbm, w2b_ref, False)

        p2 = []
        for j in range(N_DEV):
            s = (me + j) % N_DEV
            if j > 0:
                pltpu.make_async_remote_copy(
                    src_ref=send_ref.at[s],
                    dst_ref=recv_ref.at[s],
                    send_sem=s1.at[s],
                    recv_sem=r1.at[s],
                    device_id=(s,),
                    device_id_type=pl.DeviceIdType.MESH,
                ).wait_recv()
            for k in range(EPD):
                xk = recv_ref[s, k * CAP:(k + 1) * CAP, :]
                h = jnp.dot(xk, w1b_ref[k],
                            preferred_element_type=jnp.float32)
                h = jnp.maximum(h, 0.0).astype(jnp.bfloat16)
                y = jnp.dot(h, w2b_ref[k],
                            preferred_element_type=jnp.float32)
                recv_ref[s, k * CAP:(k + 1) * CAP, :] = y.astype(jnp.bfloat16)
            if j > 0:
                rdma = pltpu.make_async_remote_copy(
                    src_ref=recv_ref.at[s],
                    dst_ref=send_ref.at[me],
                    send_sem=s2.at[s],
                    recv_sem=r2.at[me],
                    device_id=(s,),
                    device_id_type=pl.DeviceIdType.MESH,
                )
                rdma.start()
                p2.append(rdma)

        iota_c = lax.broadcasted_iota(jnp.int32, (1, BLOCK), 1)
        H = T // 2
        for j in range(N_DEV):
            d = (me + j) % N_DEV
            if j > 0:
                pltpu.make_async_remote_copy(
                    src_ref=recv_ref.at[d],
                    dst_ref=send_ref.at[d],
                    send_sem=s2.at[d],
                    recv_sem=r2.at[d],
                    device_id=(d,),
                    device_id_type=pl.DeviceIdType.MESH,
                ).wait_recv()
            blk = recv_ref[d] if j == 0 else send_ref[d]
            for half in range(2):
                lo = half * H
                sd = (scol_ref[lo:lo + H, :] - d * BLOCK == iota_c).astype(
                    jnp.bfloat16
                )
                contrib = jnp.dot(
                    sd, blk, preferred_element_type=jnp.float32
                )
                if j == 0:
                    out_ref[lo:lo + H, :] = contrib
                else:
                    out_ref[lo:lo + H, :] = out_ref[lo:lo + H, :] + contrib

        for rdma in p1 + p2:
            rdma.wait_send()

    return pl.pallas_call(
        body,
        out_shape=jax.ShapeDtypeStruct((T, D), jnp.float32),
        in_specs=[
            pl.BlockSpec(memory_space=pltpu.MemorySpace.VMEM),
            pl.BlockSpec(memory_space=pltpu.MemorySpace.VMEM),
            pl.BlockSpec(memory_space=pltpu.MemorySpace.VMEM),
            pl.BlockSpec(memory_space=pltpu.MemorySpace.HBM),
            pl.BlockSpec(memory_space=pltpu.MemorySpace.HBM),
        ],
        out_specs=pl.BlockSpec(memory_space=pltpu.MemorySpace.VMEM),
        scratch_shapes=[
            pltpu.VMEM((N_DEV, BLOCK, D), jnp.bfloat16),
            pltpu.VMEM((N_DEV, BLOCK, D), jnp.bfloat16),
            pltpu.VMEM((EPD, D, F), jnp.bfloat16),
            pltpu.VMEM((EPD, F, D), jnp.bfloat16),
            pltpu.VMEM((2, D // 2, D), jnp.float32),
            pltpu.SemaphoreType.DMA((N_DEV,)),
            pltpu.SemaphoreType.DMA((N_DEV,)),
            pltpu.SemaphoreType.DMA((N_DEV,)),
            pltpu.SemaphoreType.DMA((N_DEV,)),
            pltpu.SemaphoreType.DMA((2,)),
        ],
        compiler_params=pltpu.CompilerParams(
            collective_id=0,
            vmem_limit_bytes=63 * 1024 * 1024,
        ),
    )(xb, slot_row, slot_col, W1, W2)


def kernel(x, assign, W1, W2):
    T, _ = x.shape

    xb = x.astype(jnp.bfloat16)

    e = assign.astype(jnp.int32)
    oh = (e[:, None] == jnp.arange(N_EXPERTS, dtype=jnp.int32)[None, :])
    ohi = oh.astype(jnp.int32)
    rank = jnp.sum(jnp.cumsum(ohi, axis=0) * ohi, axis=1) - 1
    slot = e * CAP + rank

    return _a2a_moe_a2a(
        xb, slot.reshape(1, T), slot.reshape(T, 1), W1, W2
    )


# device time: 122343 ns/iter; 1.0704x vs baseline; 1.0704x over previous
import jax
import jax.numpy as jnp
from jax import lax
from jax.experimental import pallas as pl
from jax.experimental.pallas import tpu as pltpu

N_DEV = 4
N_EXPERTS = 16
EPD = N_EXPERTS // N_DEV
CAP = 160
BLOCK = EPD * CAP


def _a2a_moe_a2a(xb, slot_row, slot_col, W1, W2):
    T, D = xb.shape
    F = W1.shape[2]

    def body(x_ref, srow_ref, scol_ref, w1_hbm, w2_hbm, out_ref,
             send_ref, recv_ref, w1b_ref, w2b_ref, stage_ref,
             s1, r1, s2, r2, csem):
        me = lax.axis_index("i")

        barrier = pltpu.get_barrier_semaphore()
        for j in range(1, N_DEV):
            pl.semaphore_signal(
                barrier, inc=1,
                device_id=((me + j) % N_DEV,),
                device_id_type=pl.DeviceIdType.MESH,
            )

        iota_r = lax.broadcasted_iota(jnp.int32, (BLOCK, 1), 0)

        def pack(dst):
            sd = (srow_ref[...] - dst * BLOCK == iota_r).astype(jnp.bfloat16)
            return jnp.dot(
                sd, x_ref[...], preferred_element_type=jnp.float32
            ).astype(jnp.bfloat16)

        pl.semaphore_wait(barrier, N_DEV - 1)
        p1 = []
        for j in (2, 1, 3):
            p = (me + j) % N_DEV
            send_ref[p] = pack(p)
            rdma = pltpu.make_async_remote_copy(
                src_ref=send_ref.at[p],
                dst_ref=recv_ref.at[me],
                send_sem=s1.at[p],
                recv_sem=r1.at[me],
                device_id=(p,),
                device_id_type=pl.DeviceIdType.MESH,
            )
            rdma.start()
            p1.append(rdma)

        recv_ref[me] = pack(me)

        n_chunks = EPD * 2

        def make_cast_loop(w_hbm, wb_ref, w1_layout):
            def chunk_copy(i):
                k = i // 2
                hs = pl.ds((i % 2) * (F // 2), F // 2)
                src = w_hbm.at[k, :, hs] if w1_layout else w_hbm.at[k, hs, :]
                return pltpu.make_async_copy(src, stage_ref.at[i % 2],
                                             csem.at[i % 2])

            def step(i, carry):
                @pl.when(i + 1 < n_chunks)
                def _():
                    chunk_copy(i + 1).start()
                chunk_copy(i).wait()
                k = i // 2
                hs = pl.ds((i % 2) * (F // 2), F // 2)
                val = stage_ref[i % 2].astype(jnp.bfloat16)
                if w1_layout:
                    wb_ref[k, :, hs] = val
                else:
                    wb_ref[k, hs, :] = val
                return carry

            chunk_copy(0).start()
            lax.fori_loop(0, n_chunks, step, 0)

        make_cast_loop(w1_hbm, w1b_ref, True)
        make_cast_loop(w2_hbm, w2b_ref, False)

        p2 = []
        for j in range(N_DEV):
            s = (me + j) % N_DEV
            if j > 0:
                pltpu.make_async_remote_copy(
                    src_ref=send_ref.at[s],
                    dst_ref=recv_ref.at[s],
                    send_sem=s1.at[s],
                    recv_sem=r1.at[s],
                    device_id=(s,),
                    device_id_type=pl.DeviceIdType.MESH,
                ).wait_recv()
            for k in range(EPD):
                xk = recv_ref[s, k * CAP:(k + 1) * CAP, :]
                h = jnp.dot(xk, w1b_ref[k],
                            preferred_element_type=jnp.float32)
                h = jnp.maximum(h, 0.0).astype(jnp.bfloat16)
                y = jnp.dot(h, w2b_ref[k],
                            preferred_element_type=jnp.float32)
                recv_ref[s, k * CAP:(k + 1) * CAP, :] = y.astype(jnp.bfloat16)
            if j > 0:
                rdma = pltpu.make_async_remote_copy(
                    src_ref=recv_ref.at[s],
                    dst_ref=send_ref.at[me],
                    send_sem=s2.at[s],
                    recv_sem=r2.at[me],
                    device_id=(s,),
                    device_id_type=pl.DeviceIdType.MESH,
                )
                rdma.start()
                p2.append(rdma)

        iota_c = lax.broadcasted_iota(jnp.int32, (1, BLOCK), 1)
        H = T // 2
        for j in range(N_DEV):
            d = (me + j) % N_DEV
            if j > 0:
                pltpu.make_async_remote_copy(
                    src_ref=recv_ref.at[d],
                    dst_ref=send_ref.at[d],
                    send_sem=s2.at[d],
                    recv_sem=r2.at[d],
                    device_id=(d,),
                    device_id_type=pl.DeviceIdType.MESH,
                ).wait_recv()
            blk = recv_ref[d] if j == 0 else send_ref[d]
            for half in range(2):
                lo = half * H
                sd = (scol_ref[lo:lo + H, :] - d * BLOCK == iota_c).astype(
                    jnp.bfloat16
                )
                contrib = jnp.dot(
                    sd, blk, preferred_element_type=jnp.float32
                ).astype(jnp.bfloat16)
                if j == 0:
                    out_ref[lo:lo + H, :] = contrib
                else:
                    out_ref[lo:lo + H, :] = out_ref[lo:lo + H, :] + contrib

        for rdma in p1 + p2:
            rdma.wait_send()

    return pl.pallas_call(
        body,
        out_shape=jax.ShapeDtypeStruct((T, D), jnp.bfloat16),
        in_specs=[
            pl.BlockSpec(memory_space=pltpu.MemorySpace.VMEM),
            pl.BlockSpec(memory_space=pltpu.MemorySpace.VMEM),
            pl.BlockSpec(memory_space=pltpu.MemorySpace.VMEM),
            pl.BlockSpec(memory_space=pltpu.MemorySpace.HBM),
            pl.BlockSpec(memory_space=pltpu.MemorySpace.HBM),
        ],
        out_specs=pl.BlockSpec(memory_space=pltpu.MemorySpace.VMEM),
        scratch_shapes=[
            pltpu.VMEM((N_DEV, BLOCK, D), jnp.bfloat16),
            pltpu.VMEM((N_DEV, BLOCK, D), jnp.bfloat16),
            pltpu.VMEM((EPD, D, F), jnp.bfloat16),
            pltpu.VMEM((EPD, F, D), jnp.bfloat16),
            pltpu.VMEM((2, D, D), jnp.float32),
            pltpu.SemaphoreType.DMA((N_DEV,)),
            pltpu.SemaphoreType.DMA((N_DEV,)),
            pltpu.SemaphoreType.DMA((N_DEV,)),
            pltpu.SemaphoreType.DMA((N_DEV,)),
            pltpu.SemaphoreType.DMA((2,)),
        ],
        compiler_params=pltpu.CompilerParams(
            collective_id=0,
            vmem_limit_bytes=63 * 1024 * 1024,
        ),
    )(xb, slot_row, slot_col, W1, W2)


def kernel(x, assign, W1, W2):
    T, _ = x.shape

    xb = x.astype(jnp.bfloat16)

    e = assign.astype(jnp.int32)
    oh = (e[:, None] == jnp.arange(N_EXPERTS, dtype=jnp.int32)[None, :])
    ohi = oh.astype(jnp.int32)
    rank = jnp.sum(jnp.cumsum(ohi, axis=0) * ohi, axis=1) - 1
    slot = e * CAP + rank

    out = _a2a_moe_a2a(
        xb, slot.reshape(1, T), slot.reshape(T, 1), W1, W2
    )
    return out.astype(jnp.float32)
